# baseline (speedup 1.0000x reference)
import functools

import jax
import jax.numpy as jnp
from jax import lax
from jax.experimental import pallas as pl
from jax.experimental.pallas import tpu as pltpu
from jax.experimental.pallas import tpu_sc as plsc

_B = 1024
_L = 50
_V = 1000
_D = 1000
_N = _B * _L
_NW = 32
_PER_W = _N // _NW
_C = 32
_NCH = _PER_W // _C


def _gather_body(idx_hbm, table_hbm, out_hbm, idx_v, rows_v, gsem):
    wid = lax.axis_index("s") * 2 + lax.axis_index("c")
    base = wid * _PER_W
    pltpu.sync_copy(idx_hbm.at[pl.ds(base, _PER_W)], idx_v)

    def chunk(c, carry):
        cb = c * _C
        pltpu.async_copy(
            table_hbm.at[idx_v.at[pl.ds(cb, _C)]], rows_v, gsem
        ).wait()
        pltpu.sync_copy(rows_v, out_hbm.at[pl.ds(base + cb, _C)])
        return carry

    lax.fori_loop(0, _NCH, chunk, 0)


_mesh = plsc.VectorSubcoreMesh(core_axis_name="c", subcore_axis_name="s")

_gather = functools.partial(
    pl.kernel,
    out_type=jax.ShapeDtypeStruct((_N, _D), jnp.float32),
    mesh=_mesh,
    scratch_types=[
        pltpu.VMEM((_PER_W,), jnp.int32),
        pltpu.VMEM((_C, _D), jnp.float32),
        pltpu.SemaphoreType.DMA,
    ],
    compiler_params=pltpu.CompilerParams(use_tc_tiling_on_sc=False),
)(_gather_body)


@jax.jit
def kernel(idx, token_embedding_table):
    out = _gather(idx.reshape(_N), token_embedding_table)
    return out.reshape(_B, _L, _V)


# speedup vs baseline: 1.0516x; 1.0516x over previous
import functools

import jax
import jax.numpy as jnp
from jax import lax
from jax.experimental import pallas as pl
from jax.experimental.pallas import tpu as pltpu
from jax.experimental.pallas import tpu_sc as plsc

_B = 1024
_L = 50
_V = 1000
_D = 1000
_N = _B * _L
_NW = 32
_PER_W = _N // _NW
_K = 40
_NG = _PER_W // _K
_NBUF = 2
_T = _NG // _NBUF


def _gather_body(idx_hbm, table_hbm, out_hbm, idx_v, rows_v, *sems):
    gsems = sems[:_NBUF]
    ssems = sems[_NBUF:]
    wid = lax.axis_index("s") * 2 + lax.axis_index("c")
    base = wid * _PER_W
    pltpu.sync_copy(idx_hbm.at[pl.ds(base, _PER_W)], idx_v)
    bufs = [rows_v.at[b] for b in range(_NBUF)]

    def start_gather(r, b):
        pltpu.make_async_copy(
            table_hbm.at[idx_v.at[pl.ds(r * _K, _K)]], bufs[b], gsems[b]
        ).start()

    def wait_gather(b):
        pltpu.make_async_copy(
            table_hbm.at[idx_v.at[pl.ds(0, _K)]], bufs[b], gsems[b]
        ).wait()

    def start_scatter(r, b):
        pltpu.make_async_copy(
            bufs[b], out_hbm.at[pl.ds(base + r * _K, _K)], ssems[b]
        ).start()

    def wait_scatter(b):
        pltpu.make_async_copy(
            bufs[b], out_hbm.at[pl.ds(base, _K)], ssems[b]
        ).wait()

    for b in range(_NBUF):
        start_gather(b, b)

    def round_body(t, carry):
        r0 = t * _NBUF
        for b in range(_NBUF):
            wait_gather(b)
            start_scatter(r0 + b, b)
            wait_scatter(b)
            start_gather(r0 + _NBUF + b, b)
        return carry

    lax.fori_loop(0, _T - 1, round_body, 0)

    r0 = (_T - 1) * _NBUF
    for b in range(_NBUF):
        wait_gather(b)
        start_scatter(r0 + b, b)
        wait_scatter(b)


_mesh = plsc.VectorSubcoreMesh(core_axis_name="c", subcore_axis_name="s")

_gather = functools.partial(
    pl.kernel,
    out_type=jax.ShapeDtypeStruct((_N, _D), jnp.float32),
    mesh=_mesh,
    scratch_types=[
        pltpu.VMEM((_PER_W,), jnp.int32),
        pltpu.VMEM((_NBUF, _K, _D), jnp.float32),
    ] + [pltpu.SemaphoreType.DMA] * (2 * _NBUF),
    compiler_params=pltpu.CompilerParams(use_tc_tiling_on_sc=False),
)(_gather_body)


@jax.jit
def kernel(idx, token_embedding_table):
    out = _gather(idx.reshape(_N), token_embedding_table)
    return out.reshape(_B, _L, _V)
